# baseline (speedup 1.0000x reference)
import functools

import jax
import jax.numpy as jnp
from jax import lax
from jax.experimental import pallas as pl
from jax.experimental.pallas import tpu as pltpu
from jax.experimental.pallas import tpu_sc as plsc

VOCAB = 1000000
EMBED = 32
B = 16384
L = 50
LPAD = 56
NC = 2
NS = 16
NW = NC * NS
RW = B // NW
CHUNK = LPAD
NCH = RW
NBUF = 4
WIDE = 128

_mesh = plsc.VectorSubcoreMesh(
    core_axis_name="c", subcore_axis_name="s", num_cores=NC, num_subcores=NS
)


@functools.partial(
    pl.kernel,
    out_type=jax.ShapeDtypeStruct((B * EMBED,), jnp.float32),
    mesh=_mesh,
    scratch_types=[
        pltpu.VMEM((NCH * CHUNK,), jnp.int32),
        pltpu.VMEM((NCH * 64,), jnp.int32),
        [pltpu.VMEM((CHUNK, WIDE), jnp.float32) for _ in range(NBUF)],
        pltpu.VMEM((RW * EMBED,), jnp.float32),
        [pltpu.SemaphoreType.DMA for _ in range(NBUF)],
    ],
    compiler_params=pltpu.CompilerParams(use_tc_tiling_on_sc=True),
)
def _sc_embed_mean(table_hbm, gidx_hbm, qoff_hbm, out_hbm,
                   idx_v, qoff_v, gs, out_v, sems):
    wid = lax.axis_index("c") * NS + lax.axis_index("s")
    pltpu.sync_copy(gidx_hbm.at[wid], idx_v)
    pltpu.sync_copy(qoff_hbm.at[wid], qoff_v)

    def start(c, b):
        pltpu.async_copy(
            table_hbm.at[idx_v.at[pl.ds(c * CHUNK, CHUNK)]], gs[b], sems[b])

    def wait(b):
        pltpu.make_async_copy(
            table_hbm.at[idx_v.at[pl.ds(0, CHUNK)]], gs[b], sems[b]).wait()

    scale = jnp.float32(1.0 / L)

    def process(c, b):
        g = gs[b]
        qvs = [qoff_v[pl.ds(c * 64 + t * 16, 16)] for t in range(4)]

        def q(j):
            return qvs[j // 16][j % 16]

        q0 = q(0)
        acc0 = g[0, pl.ds(q0, 16)]
        acc1 = g[0, pl.ds(q0 + 16, 16)]
        for j in range(1, L):
            qj = q(j)
            acc0 = acc0 + g[j, pl.ds(qj, 16)]
            acc1 = acc1 + g[j, pl.ds(qj + 16, 16)]
        out_v[pl.ds(c * EMBED, 16)] = acc0 * scale
        out_v[pl.ds(c * EMBED + 16, 16)] = acc1 * scale

    for b in range(NBUF):
        start(b, b)

    @pl.loop(0, NCH - NBUF, step=NBUF)
    def _(c):
        for b in range(NBUF):
            wait(b)
            process(c + b, b)
            start(c + b + NBUF, b)

    for b in range(NBUF):
        wait(b)
        process(NCH - NBUF + b, b)

    pltpu.sync_copy(out_v, out_hbm.at[pl.ds(wid * (RW * EMBED), RW * EMBED)])


def kernel(item_tensors, table):
    npad = B * (LPAD - L)
    pad_vals = (jnp.arange(npad, dtype=jnp.int32) * 97) % VOCAB
    idx = jnp.concatenate(
        [item_tensors, pad_vals.reshape(B, LPAD - L)], axis=1)
    gidx = (idx // 4).reshape(NW, NCH * CHUNK)
    qoff = jnp.pad((item_tensors % 4) * EMBED, ((0, 0), (0, 64 - L)))
    qoff = qoff.reshape(NW, NCH * 64)
    table4 = table.reshape(VOCAB // 4, WIDE)
    out = _sc_embed_mean(table4, gidx, qoff)
    return out.reshape(B, EMBED)


# speedup vs baseline: 1.1788x; 1.1788x over previous
import functools

import jax
import jax.numpy as jnp
from jax import lax
from jax.experimental import pallas as pl
from jax.experimental.pallas import tpu as pltpu
from jax.experimental.pallas import tpu_sc as plsc

VOCAB = 1000000
EMBED = 32
B = 16384
L = 50
NC = 2
NS = 16
NW = NC * NS
RW = B // NW
NCH = RW
NBUF = 4

_mesh = plsc.VectorSubcoreMesh(
    core_axis_name="c", subcore_axis_name="s", num_cores=NC, num_subcores=NS
)


@functools.partial(
    pl.kernel,
    out_type=jax.ShapeDtypeStruct((B * EMBED,), jnp.float32),
    mesh=_mesh,
    scratch_types=[
        pltpu.VMEM((NCH, L), jnp.int32),
        [pltpu.VMEM((L, EMBED), jnp.float32) for _ in range(NBUF)],
        pltpu.VMEM((RW * EMBED,), jnp.float32),
        [pltpu.SemaphoreType.DMA for _ in range(NBUF)],
    ],
    compiler_params=pltpu.CompilerParams(use_tc_tiling_on_sc=False),
)
def _sc_embed_mean(table_hbm, idx_hbm, out_hbm, idx_v, gs, out_v, sems):
    wid = lax.axis_index("c") * NS + lax.axis_index("s")
    pltpu.sync_copy(idx_hbm.at[pl.ds(wid * RW, RW)], idx_v)

    def start(c, b):
        pltpu.async_copy(table_hbm.at[idx_v.at[c]], gs[b], sems[b])

    def wait(b):
        pltpu.make_async_copy(table_hbm.at[idx_v.at[0]], gs[b], sems[b]).wait()

    scale = jnp.float32(1.0 / L)

    def process(c, b):
        g = gs[b]
        acc0 = g[0, pl.ds(0, 16)]
        acc1 = g[0, pl.ds(16, 16)]
        for j in range(1, L):
            acc0 = acc0 + g[j, pl.ds(0, 16)]
            acc1 = acc1 + g[j, pl.ds(16, 16)]
        out_v[pl.ds(c * EMBED, 16)] = acc0 * scale
        out_v[pl.ds(c * EMBED + 16, 16)] = acc1 * scale

    for b in range(NBUF):
        start(b, b)

    @pl.loop(0, NCH - NBUF, step=NBUF)
    def _(c):
        for b in range(NBUF):
            wait(b)
            process(c + b, b)
            start(c + b + NBUF, b)

    for b in range(NBUF):
        wait(b)
        process(NCH - NBUF + b, b)

    pltpu.sync_copy(out_v, out_hbm.at[pl.ds(wid * (RW * EMBED), RW * EMBED)])


def kernel(item_tensors, table):
    out = _sc_embed_mean(table, item_tensors)
    return out.reshape(B, EMBED)
